# Initial kernel scaffold; baseline (speedup 1.0000x reference)
import jax
import jax.numpy as jnp
from jax.experimental import pallas as pl


def kernel(x, edge_index, W1, w2):
    raise NotImplementedError("write your pallas kernel here")


# speedup vs baseline: 22.5347x; 22.5347x over previous
import functools

import jax
import jax.numpy as jnp
from jax import lax
from jax.experimental import pallas as pl
from jax.experimental.pallas import tpu as pltpu
from jax.experimental.pallas import tpu_sc as plsc

N = 10000
E = 320000
D = 128
L = 16
W = D + L
NC = 2
NS = 16
NW = NC * NS
EPT = E // NW
S = 80
CH = EPT // S
IB = 25
NB = CH // IB
RPS = N // NS
RB = 125
G = S // L


def _prep_body(x_ref, w1_ref, w2_ref, xp_ref, h_ref):
    xb = x_ref[...]
    t = jnp.tanh(lax.dot_general(xb, w1_ref[...], (((1,), (1,)), ((), ())),
                                 preferred_element_type=jnp.float32))
    h_ref[...] = lax.dot_general(t, w2_ref[...], (((1,), (1,)), ((), ())),
                                 preferred_element_type=jnp.float32)
    br = xb.shape[0]
    col = lax.broadcasted_iota(jnp.int32, (br, L), 1)
    extra = jnp.where(col == 0, 1.0, 0.0).astype(jnp.float32)
    xp_ref[...] = jnp.concatenate([xb, extra], axis=1)


def _prep(x, W1, w2):
    BR = 2000
    return pl.pallas_call(
        _prep_body,
        grid=(N // BR,),
        in_specs=[
            pl.BlockSpec((BR, D), lambda i: (i, 0)),
            pl.BlockSpec((D, D), lambda i: (0, 0)),
            pl.BlockSpec((1, D), lambda i: (0, 0)),
        ],
        out_specs=[
            pl.BlockSpec((BR, W), lambda i: (i, 0)),
            pl.BlockSpec((BR, 1), lambda i: (i, 0)),
        ],
        out_shape=[
            jax.ShapeDtypeStruct((N, W), jnp.float32),
            jax.ShapeDtypeStruct((N, 1), jnp.float32),
        ],
    )(x, W1, w2)


def _sc_body(xp_hbm, h_hbm, src_hbm, dst_hbm, parts_hbm,
             h_v, src_v, dst_v, idx_v, rg, wst, acc, sem):
    c = lax.axis_index("c")
    s = lax.axis_index("s")
    wid = s * NC + c

    pltpu.sync_copy(h_hbm, h_v)

    zeros = jnp.zeros((L,), jnp.float32)

    def _zrow(r, carry):
        for k in range(W // L):
            rg[r, pl.ds(k * L, L)] = zeros
        return carry

    lax.fori_loop(0, RB, _zrow, 0)
    for q in range(RPS // RB):
        pltpu.sync_copy(rg.at[pl.ds(0, RB)], acc.at[pl.ds(s * RPS + q * RB, RB)])
    plsc.subcore_barrier()

    def _chunk(ch, carry):
        def _stage(g, carry2):
            base = g * L
            idx_v[pl.ds(base, L)] = src_v[ch, pl.ds(base, L)]
            return carry2

        lax.fori_loop(0, G, _stage, 0)
        pltpu.async_copy(xp_hbm.at[idx_v], rg.at[pl.ds(0, S)], sem).wait()

        def _group(g, carry2):
            base = g * L
            src16 = idx_v[pl.ds(base, L)]
            e16 = plsc.load_gather(h_v, [src16])
            wst[pl.ds(L, L)] = jnp.exp(e16)
            for j in range(L):
                wj = plsc.load_gather(wst, [jnp.full((L,), L + j, jnp.int32)])
                r = base + j
                for k in range(W // L):
                    sl = pl.ds(k * L, L)
                    rg[r, sl] = rg[r, sl] * wj
            return carry2

        lax.fori_loop(0, G, _group, 0)
        pltpu.sync_copy(rg.at[pl.ds(0, S)], acc.at[dst_v.at[ch]], add=True)
        return carry

    def _block(blk, carry):
        pltpu.sync_copy(src_hbm.at[wid, pl.ds(blk * IB, IB)], src_v)
        pltpu.sync_copy(dst_hbm.at[wid, pl.ds(blk * IB, IB)], dst_v)
        lax.fori_loop(0, IB, _chunk, 0)
        return carry

    lax.fori_loop(0, NB, _block, 0)
    plsc.subcore_barrier()

    for q in range(RPS // RB):
        r0 = s * RPS + q * RB
        pltpu.sync_copy(acc.at[pl.ds(r0, RB)], rg)
        pltpu.sync_copy(rg, parts_hbm.at[c, pl.ds(r0, RB)])


_sc_main = functools.partial(
    pl.kernel,
    out_type=jax.ShapeDtypeStruct((NC, N, W), jnp.float32),
    mesh=plsc.VectorSubcoreMesh(core_axis_name="c", subcore_axis_name="s"),
    compiler_params=pltpu.CompilerParams(needs_layout_passes=False,
                                         use_tc_tiling_on_sc=False),
    scratch_types=[
        pltpu.VMEM((N,), jnp.float32),
        pltpu.VMEM((IB, S), jnp.int32),
        pltpu.VMEM((IB, S), jnp.int32),
        pltpu.VMEM((S,), jnp.int32),
        pltpu.VMEM((RB, W), jnp.float32),
        pltpu.VMEM((2 * L,), jnp.float32),
        pltpu.VMEM_SHARED((N, W), jnp.float32),
        pltpu.SemaphoreType.DMA,
    ],
)(_sc_body)


def _fin_body(p_ref, o_ref):
    p = p_ref[...]
    sblk = p[0] + p[1]
    num = sblk[:, :D]
    br = num.shape[0]
    col = lax.broadcasted_iota(jnp.int32, (br, W), 1)
    den = jnp.sum(jnp.where(col == D, sblk, 0.0), axis=1, keepdims=True)
    o_ref[...] = jnp.where(den > 0.0, num / den, 0.0)


def _finalize(parts):
    BR = 2000
    return pl.pallas_call(
        _fin_body,
        grid=(N // BR,),
        in_specs=[pl.BlockSpec((NC, BR, W), lambda i: (0, i, 0))],
        out_specs=pl.BlockSpec((BR, D), lambda i: (i, 0)),
        out_shape=jax.ShapeDtypeStruct((N, D), jnp.float32),
    )(parts)


def kernel(x, edge_index, W1, w2):
    xp, h = _prep(x, W1, w2)
    src = edge_index[0].reshape(NW, CH, S)
    dst = edge_index[1].reshape(NW, CH, S)
    parts = _sc_main(xp, h.reshape(N), src, dst)
    return _finalize(parts)


# speedup vs baseline: 33.5535x; 1.4890x over previous
import functools

import jax
import jax.numpy as jnp
from jax import lax
from jax.experimental import pallas as pl
from jax.experimental.pallas import tpu as pltpu
from jax.experimental.pallas import tpu_sc as plsc

N = 10000
E = 320000
D = 128
L = 16
W = D + L
NC = 2
NS = 16
NW = NC * NS
EPT = E // NW
S = 80
CH = EPT // S
IB = 25
NB = CH // IB
RPS = N // NS
RB = 125
G = S // L


def _prep_body(x_ref, w1_ref, w2_ref, xp_ref, h_ref):
    xb = x_ref[...]
    t = jnp.tanh(lax.dot_general(xb, w1_ref[...], (((1,), (1,)), ((), ())),
                                 preferred_element_type=jnp.float32))
    h_ref[...] = lax.dot_general(t, w2_ref[...], (((1,), (1,)), ((), ())),
                                 preferred_element_type=jnp.float32)
    br = xb.shape[0]
    col = lax.broadcasted_iota(jnp.int32, (br, L), 1)
    extra = jnp.where(col == 0, 1.0, 0.0).astype(jnp.float32)
    xp_ref[...] = jnp.concatenate([xb, extra], axis=1)


def _prep(x, W1, w2):
    BR = 2000
    return pl.pallas_call(
        _prep_body,
        grid=(N // BR,),
        in_specs=[
            pl.BlockSpec((BR, D), lambda i: (i, 0)),
            pl.BlockSpec((D, D), lambda i: (0, 0)),
            pl.BlockSpec((1, D), lambda i: (0, 0)),
        ],
        out_specs=[
            pl.BlockSpec((BR, W), lambda i: (i, 0)),
            pl.BlockSpec((BR, 1), lambda i: (i, 0)),
        ],
        out_shape=[
            jax.ShapeDtypeStruct((N, W), jnp.float32),
            jax.ShapeDtypeStruct((N, 1), jnp.float32),
        ],
    )(x, W1, w2)


def _sc_body(xp_hbm, h_hbm, src_hbm, dst_hbm, parts_hbm,
             h_v, src_v, dst_v, rg0, rg1, wst, acc, sem0, sem1):
    c = lax.axis_index("c")
    s = lax.axis_index("s")
    wid = s * NC + c

    pltpu.sync_copy(h_hbm, h_v)

    zeros = jnp.zeros((L,), jnp.float32)

    def _zrow(r, carry):
        for k in range(W // L):
            rg0[r, pl.ds(k * L, L)] = zeros
        return carry

    lax.fori_loop(0, S, _zrow, 0)
    r0 = s * RPS
    for q in range(RPS // S):
        pltpu.sync_copy(rg0, acc.at[pl.ds(r0 + q * S, S)])
    TAIL = RPS % S
    pltpu.sync_copy(rg0.at[pl.ds(0, TAIL)],
                    acc.at[pl.ds(r0 + RPS - TAIL, TAIL)])
    plsc.subcore_barrier()

    def _work(ch, rg):
        def _group(g, carry2):
            base = g * L
            src16 = src_v[ch, pl.ds(base, L)]
            e16 = plsc.load_gather(h_v, [src16])
            wst[pl.ds(L, L)] = jnp.exp(e16)
            for j in range(L):
                wj = plsc.load_gather(wst, [jnp.full((L,), L + j, jnp.int32)])
                r = base + j
                for k in range(W // L):
                    sl = pl.ds(k * L, L)
                    rg[r, sl] = rg[r, sl] * wj
            return carry2

        lax.fori_loop(0, G, _group, 0)
        pltpu.sync_copy(rg, acc.at[dst_v.at[ch]], add=True)

    def _block(blk, carry):
        pltpu.sync_copy(src_hbm.at[wid, pl.ds(blk * IB, IB)], src_v)
        pltpu.sync_copy(dst_hbm.at[wid, pl.ds(blk * IB, IB)], dst_v)
        pltpu.async_copy(xp_hbm.at[src_v.at[0]], rg0, sem0)

        def _pair(i, c2):
            ch0 = 2 * i
            pltpu.make_async_copy(xp_hbm.at[src_v.at[ch0]], rg0, sem0).wait()
            pltpu.async_copy(xp_hbm.at[src_v.at[ch0 + 1]], rg1, sem1)
            _work(ch0, rg0)
            pltpu.make_async_copy(xp_hbm.at[src_v.at[ch0 + 1]], rg1, sem1).wait()
            pltpu.async_copy(xp_hbm.at[src_v.at[ch0 + 2]], rg0, sem0)
            _work(ch0 + 1, rg1)
            return c2

        lax.fori_loop(0, (IB - 1) // 2, _pair, 0)
        pltpu.make_async_copy(xp_hbm.at[src_v.at[IB - 1]], rg0, sem0).wait()
        _work(IB - 1, rg0)
        return carry

    lax.fori_loop(0, NB, _block, 0)
    plsc.subcore_barrier()

    for q in range(RPS // S):
        pltpu.sync_copy(acc.at[pl.ds(r0 + q * S, S)], rg0)
        pltpu.sync_copy(rg0, parts_hbm.at[c, pl.ds(r0 + q * S, S)])
    pltpu.sync_copy(acc.at[pl.ds(r0 + RPS - TAIL, TAIL)], rg0.at[pl.ds(0, TAIL)])
    pltpu.sync_copy(rg0.at[pl.ds(0, TAIL)],
                    parts_hbm.at[c, pl.ds(r0 + RPS - TAIL, TAIL)])


_sc_main = functools.partial(
    pl.kernel,
    out_type=jax.ShapeDtypeStruct((NC, N, W), jnp.float32),
    mesh=plsc.VectorSubcoreMesh(core_axis_name="c", subcore_axis_name="s"),
    compiler_params=pltpu.CompilerParams(needs_layout_passes=False,
                                         use_tc_tiling_on_sc=False),
    scratch_types=[
        pltpu.VMEM((N,), jnp.float32),
        pltpu.VMEM((IB, S), jnp.int32),
        pltpu.VMEM((IB, S), jnp.int32),
        pltpu.VMEM((S, W), jnp.float32),
        pltpu.VMEM((S, W), jnp.float32),
        pltpu.VMEM((2 * L,), jnp.float32),
        pltpu.VMEM_SHARED((N, W), jnp.float32),
        pltpu.SemaphoreType.DMA,
        pltpu.SemaphoreType.DMA,
    ],
)(_sc_body)


def _fin_body(p_ref, o_ref):
    p = p_ref[...]
    sblk = p[0] + p[1]
    num = sblk[:, :D]
    br = num.shape[0]
    col = lax.broadcasted_iota(jnp.int32, (br, W), 1)
    den = jnp.sum(jnp.where(col == D, sblk, 0.0), axis=1, keepdims=True)
    o_ref[...] = jnp.where(den > 0.0, num / den, 0.0)


def _finalize(parts):
    BR = 2000
    return pl.pallas_call(
        _fin_body,
        grid=(N // BR,),
        in_specs=[pl.BlockSpec((NC, BR, W), lambda i: (0, i, 0))],
        out_specs=pl.BlockSpec((BR, D), lambda i: (i, 0)),
        out_shape=jax.ShapeDtypeStruct((N, D), jnp.float32),
    )(parts)


def kernel(x, edge_index, W1, w2):
    xp, h = _prep(x, W1, w2)
    src = edge_index[0].reshape(NW, CH, S)
    dst = edge_index[1].reshape(NW, CH, S)
    parts = _sc_main(xp, h.reshape(N), src, dst)
    return _finalize(parts)
